# baseline (speedup 1.0000x reference)
import functools

import jax
import jax.numpy as jnp
from jax import lax
from jax.experimental import pallas as pl
from jax.experimental.pallas import tpu as pltpu
from jax.experimental.pallas import tpu_sc as plsc

N = 10000
E = 320000
G = 64
D_IN = 128
H = 64
C = 2

NC = 2
NS = 16
NPAD = 10240
EPT = 10240
EPAD = NC * NS * EPT
CH = 128
NCHUNK = EPT // CH
TOTCH = EPAD // CH
N0 = 128
N1 = TOTCH // NS - N0
ZROWS = NPAD // NS

_mesh = plsc.VectorSubcoreMesh(core_axis_name="c", subcore_axis_name="s")
_sc_params = pltpu.CompilerParams(use_tc_tiling_on_sc=False,
                                  needs_layout_passes=False)


def _deg_body(dst_hbm, out_hbm, didx, ones_v, zbuf, acc_sh, sem):
    c = lax.axis_index("c")
    s = lax.axis_index("s")
    wid = c * NS + s
    one16 = jnp.full((16,), 1.0, jnp.float32)
    zero16 = jnp.zeros((16,), jnp.float32)

    def fill(i, _):
        ones_v[i, :] = one16
        return 0
    lax.fori_loop(0, CH, fill, 0)

    def zfill(i, _):
        zbuf[i, :] = zero16
        return 0
    lax.fori_loop(0, ZROWS, zfill, 0)

    pltpu.sync_copy(zbuf, acc_sh.at[pl.ds(s * ZROWS, ZROWS)])
    pltpu.sync_copy(dst_hbm.at[pl.ds(wid * NCHUNK, NCHUNK)], didx)
    plsc.subcore_barrier()

    def step(g, _):
        for u in range(8):
            pltpu.async_copy(ones_v, acc_sh.at[didx.at[8 * g + u]], sem,
                             add=True)
        for u in range(8):
            pltpu.make_async_copy(
                ones_v, acc_sh.at[didx.at[8 * g + u]], sem).wait()
        return 0
    lax.fori_loop(0, NCHUNK // 8, step, 0)

    plsc.subcore_barrier()
    rows = pl.ds(s * ZROWS, ZROWS)
    pltpu.sync_copy(acc_sh.at[rows], zbuf)
    pltpu.sync_copy(zbuf, out_hbm.at[c, rows])


@functools.partial(jax.jit, static_argnums=())
def _sc_deg(dstp):
    k = pl.kernel(
        _deg_body,
        out_type=jax.ShapeDtypeStruct((NC, NPAD, 16), jnp.float32),
        mesh=_mesh,
        compiler_params=_sc_params,
        scratch_types=[
            pltpu.VMEM((NCHUNK, CH), jnp.int32),
            pltpu.VMEM((CH, 16), jnp.float32),
            pltpu.VMEM((ZROWS, 16), jnp.float32),
            pltpu.VMEM_SHARED((NPAD, 16), jnp.float32),
            pltpu.SemaphoreType.DMA,
        ],
    )
    return k(dstp)


def _make_agg_body(layer1):
    def body(src_hbm, dst_hbm, xp_hbm, aux_hbm, *rest):
        if layer1:
            (out_hbm, dinv_out, sidx, didx, gb0, gb1, zbuf, dbuf, dbuf2,
             acc_sh, xs_sh, sg0, sg1) = rest
        else:
            (out_hbm, sidx, didx, gb0, gb1, zbuf, dbuf, dbuf2,
             acc_sh, xs_sh, sg0, sg1) = rest
        gbs = (gb0, gb1)
        sgs = (sg0, sg1)
        c = lax.axis_index("c")
        s = lax.axis_index("s")
        wid = c * NS + s
        zero16 = jnp.zeros((16,), jnp.float32)

        cb = wid * NCHUNK
        pltpu.async_copy(src_hbm.at[pl.ds(cb, NCHUNK)], sidx, sg0)
        pltpu.async_copy(dst_hbm.at[pl.ds(cb, NCHUNK)], didx, sg1)

        def zfill(i, _):
            for j in range(4):
                zbuf[i, pl.ds(j * 16, 16)] = zero16
            return 0
        lax.fori_loop(0, CH, zfill, 0)

        def stage(i, _):
            rows = pl.ds(s * ZROWS + i * CH, CH)
            pltpu.sync_copy(zbuf, acc_sh.at[rows])
            if layer1:
                pltpu.sync_copy(aux_hbm.at[0, rows], dbuf)
                pltpu.sync_copy(aux_hbm.at[1, rows], dbuf2)

                def newt(r2, _):
                    for u in range(4):
                        r = 4 * r2 + u
                        d = dbuf[r, :] + dbuf2[r, :] + 1.0
                        iv = plsc.bitcast(d, jnp.int32)
                        iv = 0x5F3759DF - lax.shift_right_logical(iv, 1)
                        y = plsc.bitcast(iv, jnp.float32)
                        for _ in range(3):
                            y = y * (1.5 - 0.5 * d * y * y)
                        dbuf[r, :] = y
                    return 0
                lax.fori_loop(0, CH // 4, newt, 0)
            else:
                pltpu.sync_copy(aux_hbm.at[rows], dbuf)
            pltpu.sync_copy(xp_hbm.at[rows], gb0)

            def scale(r2, _):
                for u in range(2):
                    r = 2 * r2 + u
                    dv = dbuf[r, :]
                    for j in range(4):
                        sl = pl.ds(j * 16, 16)
                        gb0[r, sl] = gb0[r, sl] * dv
                return 0
            lax.fori_loop(0, CH // 2, scale, 0)
            pltpu.sync_copy(gb0, xs_sh.at[rows])
            if layer1:
                @pl.when(c == 0)
                def _():
                    pltpu.sync_copy(dbuf, dinv_out.at[rows])
            return 0
        lax.fori_loop(0, ZROWS // CH, stage, 0)

        pltpu.make_async_copy(
            src_hbm.at[pl.ds(cb, NCHUNK)], sidx, sg0).wait()
        pltpu.make_async_copy(
            dst_hbm.at[pl.ds(cb, NCHUNK)], didx, sg1).wait()
        plsc.subcore_barrier()

        pltpu.async_copy(xs_sh.at[sidx.at[0]], gb0, sg0)

        def body_loop(g, _):
            k0 = 2 * g
            for b in range(2):
                k = k0 + b
                pltpu.make_async_copy(
                    xs_sh.at[sidx.at[k]], gbs[b], sgs[b]).wait()

                @pl.when(k + 1 < NCHUNK)
                def _():
                    pltpu.async_copy(xs_sh.at[sidx.at[k + 1]],
                                     gbs[1 - b], sgs[1 - b])

                pltpu.sync_copy(gbs[b], acc_sh.at[didx.at[k]], add=True)
            return 0
        lax.fori_loop(0, NCHUNK // 2, body_loop, 0)

        plsc.subcore_barrier()

        def wb(i, _):
            rows = pl.ds(s * ZROWS + i * CH, CH)
            pltpu.sync_copy(acc_sh.at[rows], zbuf)
            pltpu.sync_copy(zbuf, out_hbm.at[c, rows])
            return 0
        lax.fori_loop(0, ZROWS // CH, wb, 0)
    return body


_agg1_body = _make_agg_body(True)
_agg2_body = _make_agg_body(False)

_AGG_SCRATCH = [
    pltpu.VMEM((NCHUNK, CH), jnp.int32),
    pltpu.VMEM((NCHUNK, CH), jnp.int32),
    pltpu.VMEM((CH, H), jnp.float32),
    pltpu.VMEM((CH, H), jnp.float32),
    pltpu.VMEM((CH, H), jnp.float32),
    pltpu.VMEM((CH, 16), jnp.float32),
    pltpu.VMEM((CH, 16), jnp.float32),
    pltpu.VMEM_SHARED((NPAD, H), jnp.float32),
    pltpu.VMEM_SHARED((NPAD, H), jnp.float32),
    pltpu.SemaphoreType.DMA,
    pltpu.SemaphoreType.DMA,
]


def _sc_agg1(srcp, dstp, xp, degp):
    k = pl.kernel(
        _agg1_body,
        out_type=(jax.ShapeDtypeStruct((NC, NPAD, H), jnp.float32),
                  jax.ShapeDtypeStruct((NPAD, 16), jnp.float32)),
        mesh=_mesh,
        compiler_params=_sc_params,
        scratch_types=_AGG_SCRATCH,
    )
    return k(srcp, dstp, xp, degp)


def _sc_agg2(srcp, dstp, xp, dinv16):
    k = pl.kernel(
        _agg2_body,
        out_type=jax.ShapeDtypeStruct((NC, NPAD, H), jnp.float32),
        mesh=_mesh,
        compiler_params=_sc_params,
        scratch_types=_AGG_SCRATCH,
    )
    return k(srcp, dstp, xp, dinv16)


_BLK = 512
_NBLK = NPAD // _BLK


def _tc0_body(x_ref, w1_ref, xp_ref):
    xp_ref[...] = jnp.dot(x_ref[...], w1_ref[...],
                          preferred_element_type=jnp.float32)


def _tc0(xpad, W1):
    return pl.pallas_call(
        _tc0_body,
        grid=(_NBLK,),
        in_specs=[
            pl.BlockSpec((_BLK, D_IN), lambda i: (i, 0)),
            pl.BlockSpec((D_IN, H), lambda i: (0, 0)),
        ],
        out_specs=pl.BlockSpec((_BLK, H), lambda i: (i, 0)),
        out_shape=jax.ShapeDtypeStruct((NPAD, H), jnp.float32),
    )(xpad, W1)


def _tc2_body(acc_ref, xp_ref, dinv_ref, b1_ref, w2_ref, xp2_ref):
    dinv = dinv_ref[:, 0:1]
    agg = acc_ref[0] + acc_ref[1]
    h1 = jnp.maximum(dinv * agg + (dinv * dinv) * xp_ref[...] + b1_ref[...],
                     0.0)
    xp2_ref[...] = jnp.dot(h1, w2_ref[...], preferred_element_type=jnp.float32)


def _tc2(acc1, xp1, dinv16, b1, W2):
    return pl.pallas_call(
        _tc2_body,
        grid=(_NBLK,),
        in_specs=[
            pl.BlockSpec((NC, _BLK, H), lambda i: (0, i, 0)),
            pl.BlockSpec((_BLK, H), lambda i: (i, 0)),
            pl.BlockSpec((_BLK, 16), lambda i: (i, 0)),
            pl.BlockSpec((1, H), lambda i: (0, 0)),
            pl.BlockSpec((H, H), lambda i: (0, 0)),
        ],
        out_specs=pl.BlockSpec((_BLK, H), lambda i: (i, 0)),
        out_shape=jax.ShapeDtypeStruct((NPAD, H), jnp.float32),
    )(acc1, xp1, dinv16, b1, W2)


def _tc3_body(acc_ref, xp_ref, dinv_ref, b2_ref, batch_ref,
              l1w_ref, l1b_ref, l2w_ref, l2b_ref, out_ref, psum, cnt):
    i = pl.program_id(0)
    dinv = dinv_ref[:, 0:1]
    agg = acc_ref[0] + acc_ref[1]
    h2 = jnp.maximum(dinv * agg + (dinv * dinv) * xp_ref[...] + b2_ref[...],
                     0.0)
    seg = lax.broadcasted_iota(jnp.int32, (1, G), 1)
    p = (batch_ref[...] == seg).astype(jnp.float32)
    dn = (((0,), (0,)), ((), ()))
    ps = lax.dot_general(p, h2, dn, preferred_element_type=jnp.float32)
    cs = lax.dot_general(p, jnp.ones((_BLK, 1), jnp.float32), dn,
                         preferred_element_type=jnp.float32)

    @pl.when(i == 0)
    def _():
        psum[...] = ps
        cnt[...] = cs

    @pl.when(i > 0)
    def _():
        psum[...] += ps
        cnt[...] += cs

    @pl.when(i == _NBLK - 1)
    def _():
        pooled = psum[...] / jnp.maximum(cnt[...], 1.0)
        t = jnp.maximum(
            jnp.dot(pooled, l1w_ref[...], preferred_element_type=jnp.float32)
            + l1b_ref[...], 0.0)
        logits = (jnp.dot(t, l2w_ref[...], preferred_element_type=jnp.float32)
                  + l2b_ref[...])
        m = jnp.max(logits, axis=1, keepdims=True)
        lse = jnp.log(jnp.sum(jnp.exp(logits - m), axis=1, keepdims=True))
        out_ref[...] = logits - m - lse


def _tc3(acc2, xp2, dinv16, b2, batchp, L1w, L1b, L2w, L2b):
    return pl.pallas_call(
        _tc3_body,
        grid=(_NBLK,),
        in_specs=[
            pl.BlockSpec((NC, _BLK, H), lambda i: (0, i, 0)),
            pl.BlockSpec((_BLK, H), lambda i: (i, 0)),
            pl.BlockSpec((_BLK, 16), lambda i: (i, 0)),
            pl.BlockSpec((1, H), lambda i: (0, 0)),
            pl.BlockSpec((_BLK, 1), lambda i: (i, 0)),
            pl.BlockSpec((H, 32), lambda i: (0, 0)),
            pl.BlockSpec((1, 32), lambda i: (0, 0)),
            pl.BlockSpec((32, C), lambda i: (0, 0)),
            pl.BlockSpec((1, C), lambda i: (0, 0)),
        ],
        out_specs=pl.BlockSpec((G, C), lambda i: (0, 0)),
        out_shape=jax.ShapeDtypeStruct((G, C), jnp.float32),
        scratch_shapes=[
            pltpu.VMEM((G, H), jnp.float32),
            pltpu.VMEM((G, 1), jnp.float32),
        ],
    )(acc2, xp2, dinv16, b2, batchp, L1w, L1b, L2w, L2b)


def kernel(x, edge_index, batch, W1, b1, W2, b2, L1w, L1b, L2w, L2b):
    pad_e = EPAD - E
    srcp = jnp.concatenate(
        [edge_index[0], jnp.full((pad_e,), N, jnp.int32)]
    ).reshape(TOTCH, CH)
    dstp = jnp.concatenate(
        [edge_index[1], jnp.full((pad_e,), N, jnp.int32)]
    ).reshape(TOTCH, CH)
    xpad = jnp.pad(x, ((0, NPAD - N), (0, 0)))
    batchp = jnp.concatenate(
        [batch, jnp.full((NPAD - N,), G, jnp.int32)]).reshape(NPAD, 1)

    degp = _sc_deg(dstp)
    xp1 = _tc0(xpad, W1)
    acc1, dinv16 = _sc_agg1(srcp, dstp, xp1, degp)
    xp2 = _tc2(acc1, xp1, dinv16, b1.reshape(1, H), W2)
    acc2 = _sc_agg2(srcp, dstp, xp2, dinv16)
    return _tc3(acc2, xp2, dinv16, b2.reshape(1, H), batchp,
                L1w, L1b.reshape(1, 32), L2w, L2b.reshape(1, C))


# speedup vs baseline: 1.0172x; 1.0172x over previous
import jax
import jax.numpy as jnp
from jax import lax
from jax.experimental import pallas as pl
from jax.experimental.pallas import tpu as pltpu
from jax.experimental.pallas import tpu_sc as plsc

N = 10000
E = 320000
G = 64
D_IN = 128
H = 64
C = 2

NC = 2
NS = 16
NPAD = 10240
EPT = 10240
EPAD = NC * NS * EPT
CH = 128
NCHUNK = EPT // CH
TOTCH = EPAD // CH
ZROWS = NPAD // NS

_mesh = plsc.VectorSubcoreMesh(core_axis_name="c", subcore_axis_name="s")
_sc_params = pltpu.CompilerParams(use_tc_tiling_on_sc=False,
                                  needs_layout_passes=False)


def _deg_body(dst_hbm, out_hbm, didx, ones_v, zbuf, acc_sh, sem):
    c = lax.axis_index("c")
    s = lax.axis_index("s")
    wid = c * NS + s
    one16 = jnp.full((16,), 1.0, jnp.float32)
    zero16 = jnp.zeros((16,), jnp.float32)

    def fill(i, _):
        ones_v[i, :] = one16
        return 0
    lax.fori_loop(0, CH, fill, 0)

    def zfill(i, _):
        zbuf[i, :] = zero16
        return 0
    lax.fori_loop(0, ZROWS, zfill, 0)

    pltpu.sync_copy(zbuf, acc_sh.at[pl.ds(s * ZROWS, ZROWS)])
    pltpu.sync_copy(dst_hbm.at[pl.ds(wid * NCHUNK, NCHUNK)], didx)
    plsc.subcore_barrier()

    def step(g, _):
        for u in range(8):
            pltpu.async_copy(ones_v, acc_sh.at[didx.at[8 * g + u]], sem,
                             add=True)
        for u in range(8):
            pltpu.make_async_copy(
                ones_v, acc_sh.at[didx.at[8 * g + u]], sem).wait()
        return 0
    lax.fori_loop(0, NCHUNK // 8, step, 0)

    plsc.subcore_barrier()
    rows = pl.ds(s * ZROWS, ZROWS)
    pltpu.sync_copy(acc_sh.at[rows], zbuf)
    pltpu.sync_copy(zbuf, out_hbm.at[c, rows])


def _sc_deg(dstp):
    k = pl.kernel(
        _deg_body,
        out_type=jax.ShapeDtypeStruct((NC, NPAD, 16), jnp.float32),
        mesh=_mesh,
        compiler_params=_sc_params,
        scratch_types=[
            pltpu.VMEM((NCHUNK, CH), jnp.int32),
            pltpu.VMEM((CH, 16), jnp.float32),
            pltpu.VMEM((ZROWS, 16), jnp.float32),
            pltpu.VMEM_SHARED((NPAD, 16), jnp.float32),
            pltpu.SemaphoreType.DMA,
        ],
    )
    return k(dstp)


def _make_agg_body(layer1):
    def body(src_hbm, dst_hbm, xp_hbm, aux_hbm, *rest):
        if layer1:
            (out_hbm, dinv_out, sidx, didx, gb0, gb1, zbuf, dbuf, dbuf2,
             acc_sh, xs_sh, sg0, sg1) = rest
        else:
            (out_hbm, sidx, didx, gb0, gb1, zbuf, dbuf, dbuf2,
             acc_sh, xs_sh, sg0, sg1) = rest
        gbs = (gb0, gb1)
        sgs = (sg0, sg1)
        c = lax.axis_index("c")
        s = lax.axis_index("s")
        wid = c * NS + s
        zero16 = jnp.zeros((16,), jnp.float32)

        cb = wid * NCHUNK
        pltpu.async_copy(src_hbm.at[pl.ds(cb, NCHUNK)], sidx, sg0)
        pltpu.async_copy(dst_hbm.at[pl.ds(cb, NCHUNK)], didx, sg1)

        def zfill(i, _):
            for j in range(4):
                zbuf[i, pl.ds(j * 16, 16)] = zero16
            return 0
        lax.fori_loop(0, CH, zfill, 0)

        def stage(i, _):
            rows = pl.ds(s * ZROWS + i * CH, CH)
            pltpu.sync_copy(zbuf, acc_sh.at[rows])
            if layer1:
                pltpu.sync_copy(aux_hbm.at[0, rows], dbuf)
                pltpu.sync_copy(aux_hbm.at[1, rows], dbuf2)

                def newt(r2, _):
                    for u in range(4):
                        r = 4 * r2 + u
                        d = dbuf[r, :] + dbuf2[r, :] + 1.0
                        iv = plsc.bitcast(d, jnp.int32)
                        iv = 0x5F3759DF - lax.shift_right_logical(iv, 1)
                        y = plsc.bitcast(iv, jnp.float32)
                        for _ in range(3):
                            y = y * (1.5 - 0.5 * d * y * y)
                        dbuf[r, :] = y
                    return 0
                lax.fori_loop(0, CH // 4, newt, 0)
            else:
                pltpu.sync_copy(aux_hbm.at[rows], dbuf)
            pltpu.sync_copy(xp_hbm.at[rows], gb0)

            def scale(r2, _):
                for u in range(2):
                    r = 2 * r2 + u
                    dv = dbuf[r, :]
                    for j in range(4):
                        sl = pl.ds(j * 16, 16)
                        gb0[r, sl] = gb0[r, sl] * dv
                return 0
            lax.fori_loop(0, CH // 2, scale, 0)
            pltpu.sync_copy(gb0, xs_sh.at[rows])
            if layer1:
                @pl.when(c == 0)
                def _():
                    pltpu.sync_copy(dbuf, dinv_out.at[rows])
            return 0
        lax.fori_loop(0, ZROWS // CH, stage, 0)

        pltpu.make_async_copy(
            src_hbm.at[pl.ds(cb, NCHUNK)], sidx, sg0).wait()
        pltpu.make_async_copy(
            dst_hbm.at[pl.ds(cb, NCHUNK)], didx, sg1).wait()
        plsc.subcore_barrier()

        pltpu.async_copy(xs_sh.at[sidx.at[0]], gb0, sg0)

        def body_loop(g, _):
            k0 = 2 * g
            for b in range(2):
                k = k0 + b
                pltpu.make_async_copy(
                    xs_sh.at[sidx.at[k]], gbs[b], sgs[b]).wait()

                @pl.when(k + 1 < NCHUNK)
                def _():
                    pltpu.async_copy(xs_sh.at[sidx.at[k + 1]],
                                     gbs[1 - b], sgs[1 - b])

                pltpu.sync_copy(gbs[b], acc_sh.at[didx.at[k]], add=True)
            return 0
        lax.fori_loop(0, NCHUNK // 2, body_loop, 0)

        plsc.subcore_barrier()

        def wb(i, _):
            rows = pl.ds(s * ZROWS + i * CH, CH)
            pltpu.sync_copy(acc_sh.at[rows], zbuf)
            pltpu.sync_copy(zbuf, out_hbm.at[c, rows])
            return 0
        lax.fori_loop(0, ZROWS // CH, wb, 0)
    return body


_agg1_body = _make_agg_body(True)
_agg2_body = _make_agg_body(False)

_AGG_SCRATCH = [
    pltpu.VMEM((NCHUNK, CH), jnp.int32),
    pltpu.VMEM((NCHUNK, CH), jnp.int32),
    pltpu.VMEM((CH, H), jnp.float32),
    pltpu.VMEM((CH, H), jnp.float32),
    pltpu.VMEM((CH, H), jnp.float32),
    pltpu.VMEM((CH, 16), jnp.float32),
    pltpu.VMEM((CH, 16), jnp.float32),
    pltpu.VMEM_SHARED((NPAD, H), jnp.float32),
    pltpu.VMEM_SHARED((NPAD, H), jnp.float32),
    pltpu.SemaphoreType.DMA,
    pltpu.SemaphoreType.DMA,
]


def _sc_agg1(srcp, dstp, xp, degp):
    k = pl.kernel(
        _agg1_body,
        out_type=(jax.ShapeDtypeStruct((NC, NPAD, H), jnp.float32),
                  jax.ShapeDtypeStruct((NPAD, 16), jnp.float32)),
        mesh=_mesh,
        compiler_params=_sc_params,
        scratch_types=_AGG_SCRATCH,
    )
    return k(srcp, dstp, xp, degp)


def _sc_agg2(srcp, dstp, xp, dinv16):
    k = pl.kernel(
        _agg2_body,
        out_type=jax.ShapeDtypeStruct((NC, NPAD, H), jnp.float32),
        mesh=_mesh,
        compiler_params=_sc_params,
        scratch_types=_AGG_SCRATCH,
    )
    return k(srcp, dstp, xp, dinv16)


_BLK = 512
_NBLK = NPAD // _BLK


def _tc0_body(x_ref, w1_ref, xp_ref):
    xp_ref[...] = jnp.dot(x_ref[...], w1_ref[...],
                          preferred_element_type=jnp.float32)


def _tc0(xpad, W1):
    return pl.pallas_call(
        _tc0_body,
        grid=(_NBLK,),
        in_specs=[
            pl.BlockSpec((_BLK, D_IN), lambda i: (i, 0)),
            pl.BlockSpec((D_IN, H), lambda i: (0, 0)),
        ],
        out_specs=pl.BlockSpec((_BLK, H), lambda i: (i, 0)),
        out_shape=jax.ShapeDtypeStruct((NPAD, H), jnp.float32),
    )(xpad, W1)


def _tc2_body(acc_ref, xp_ref, dinv_ref, b1_ref, w2_ref, xp2_ref):
    dinv = dinv_ref[:, 0:1]
    agg = acc_ref[0] + acc_ref[1]
    h1 = jnp.maximum(dinv * agg + (dinv * dinv) * xp_ref[...] + b1_ref[...],
                     0.0)
    xp2_ref[...] = jnp.dot(h1, w2_ref[...], preferred_element_type=jnp.float32)


def _tc2(acc1, xp1, dinv16, b1, W2):
    return pl.pallas_call(
        _tc2_body,
        grid=(_NBLK,),
        in_specs=[
            pl.BlockSpec((NC, _BLK, H), lambda i: (0, i, 0)),
            pl.BlockSpec((_BLK, H), lambda i: (i, 0)),
            pl.BlockSpec((_BLK, 16), lambda i: (i, 0)),
            pl.BlockSpec((1, H), lambda i: (0, 0)),
            pl.BlockSpec((H, H), lambda i: (0, 0)),
        ],
        out_specs=pl.BlockSpec((_BLK, H), lambda i: (i, 0)),
        out_shape=jax.ShapeDtypeStruct((NPAD, H), jnp.float32),
    )(acc1, xp1, dinv16, b1, W2)


def _tc3_body(acc_ref, xp_ref, dinv_ref, b2_ref, batch_ref,
              l1w_ref, l1b_ref, l2w_ref, l2b_ref, out_ref):
    dinv = dinv_ref[:, 0:1]
    agg = acc_ref[0] + acc_ref[1]
    h2 = jnp.maximum(dinv * agg + (dinv * dinv) * xp_ref[...] + b2_ref[...],
                     0.0)
    seg = lax.broadcasted_iota(jnp.int32, (1, G), 1)
    p = (batch_ref[...] == seg).astype(jnp.float32)
    pooled_sum = lax.dot_general(
        p, h2, (((0,), (0,)), ((), ())), preferred_element_type=jnp.float32)
    counts = jnp.sum(p, axis=0)
    pooled = pooled_sum / jnp.maximum(counts, 1.0)[:, None]
    t = jnp.maximum(
        jnp.dot(pooled, l1w_ref[...], preferred_element_type=jnp.float32)
        + l1b_ref[...], 0.0)
    logits = (jnp.dot(t, l2w_ref[...], preferred_element_type=jnp.float32)
              + l2b_ref[...])
    m = jnp.max(logits, axis=1, keepdims=True)
    lse = jnp.log(jnp.sum(jnp.exp(logits - m), axis=1, keepdims=True))
    out_ref[...] = logits - m - lse


def _tc3(acc2, xp2, dinv16, b2, batchp, L1w, L1b, L2w, L2b):
    return pl.pallas_call(
        _tc3_body,
        out_shape=jax.ShapeDtypeStruct((G, C), jnp.float32),
    )(acc2, xp2, dinv16, b2, batchp, L1w, L1b, L2w, L2b)


def kernel(x, edge_index, batch, W1, b1, W2, b2, L1w, L1b, L2w, L2b):
    pad_e = EPAD - E
    srcp = jnp.concatenate(
        [edge_index[0], jnp.full((pad_e,), N, jnp.int32)]
    ).reshape(TOTCH, CH)
    dstp = jnp.concatenate(
        [edge_index[1], jnp.full((pad_e,), N, jnp.int32)]
    ).reshape(TOTCH, CH)
    xpad = jnp.pad(x, ((0, NPAD - N), (0, 0)))
    batchp = jnp.concatenate(
        [batch, jnp.full((NPAD - N,), G, jnp.int32)]).reshape(NPAD, 1)

    degp = _sc_deg(dstp)
    xp1 = _tc0(xpad, W1)
    acc1, dinv16 = _sc_agg1(srcp, dstp, xp1, degp)
    xp2 = _tc2(acc1, xp1, dinv16, b1.reshape(1, H), W2)
    acc2 = _sc_agg2(srcp, dstp, xp2, dinv16)
    return _tc3(acc2, xp2, dinv16, b2.reshape(1, H), batchp,
                L1w, L1b.reshape(1, 32), L2w, L2b.reshape(1, C))


# speedup vs baseline: 1.0298x; 1.0124x over previous
import jax
import jax.numpy as jnp
from jax import lax
from jax.experimental import pallas as pl
from jax.experimental.pallas import tpu as pltpu
from jax.experimental.pallas import tpu_sc as plsc

N = 10000
E = 320000
G = 64
D_IN = 128
H = 64
C = 2

NC = 2
NS = 16
NPAD = 10240
EPT = 10240
EPAD = NC * NS * EPT
CH = 128
NCHUNK = EPT // CH
TOTCH = EPAD // CH
ZROWS = NPAD // NS

_mesh = plsc.VectorSubcoreMesh(core_axis_name="c", subcore_axis_name="s")
_sc_params = pltpu.CompilerParams(use_tc_tiling_on_sc=False,
                                  needs_layout_passes=False)


def _deg_body(dst_hbm, out_hbm, didx, ones_v, zbuf, acc_sh, sem):
    c = lax.axis_index("c")
    s = lax.axis_index("s")
    wid = c * NS + s
    one16 = jnp.full((16,), 1.0, jnp.float32)
    zero16 = jnp.zeros((16,), jnp.float32)

    def fill(i, _):
        ones_v[i, :] = one16
        return 0
    lax.fori_loop(0, CH, fill, 0)

    def zfill(i, _):
        zbuf[i, :] = zero16
        return 0
    lax.fori_loop(0, ZROWS, zfill, 0)

    pltpu.sync_copy(zbuf, acc_sh.at[pl.ds(s * ZROWS, ZROWS)])
    pltpu.sync_copy(dst_hbm.at[pl.ds(wid * NCHUNK, NCHUNK)], didx)
    plsc.subcore_barrier()

    def step(g, _):
        for u in range(8):
            pltpu.async_copy(ones_v, acc_sh.at[didx.at[8 * g + u]], sem,
                             add=True)
        for u in range(8):
            pltpu.make_async_copy(
                ones_v, acc_sh.at[didx.at[8 * g + u]], sem).wait()
        return 0
    lax.fori_loop(0, NCHUNK // 8, step, 0)

    plsc.subcore_barrier()
    rows = pl.ds(s * ZROWS, ZROWS)
    pltpu.sync_copy(acc_sh.at[rows], zbuf)
    pltpu.sync_copy(zbuf, out_hbm.at[c, rows])


def _sc_deg(dstp):
    k = pl.kernel(
        _deg_body,
        out_type=jax.ShapeDtypeStruct((NC, NPAD, 16), jnp.float32),
        mesh=_mesh,
        compiler_params=_sc_params,
        scratch_types=[
            pltpu.VMEM((NCHUNK, CH), jnp.int32),
            pltpu.VMEM((CH, 16), jnp.float32),
            pltpu.VMEM((ZROWS, 16), jnp.float32),
            pltpu.VMEM_SHARED((NPAD, 16), jnp.float32),
            pltpu.SemaphoreType.DMA,
        ],
    )
    return k(dstp)


def _make_agg_body(layer1):
    def body(src_hbm, dst_hbm, xp_hbm, aux_hbm, *rest):
        if layer1:
            (out_hbm, dinv_out, sidx, didx, gb0, gb1, gb2, dbuf, dbuf2,
             acc_sh, xs_sh, sg0, sg1, sg2) = rest
        else:
            (out_hbm, sidx, didx, gb0, gb1, gb2, dbuf, dbuf2,
             acc_sh, xs_sh, sg0, sg1, sg2) = rest
        gbs = (gb0, gb1, gb2)
        sgs = (sg0, sg1, sg2)
        c = lax.axis_index("c")
        s = lax.axis_index("s")
        wid = c * NS + s
        zero16 = jnp.zeros((16,), jnp.float32)

        cb = wid * NCHUNK
        pltpu.async_copy(src_hbm.at[pl.ds(cb, NCHUNK)], sidx, sg0)
        pltpu.async_copy(dst_hbm.at[pl.ds(cb, NCHUNK)], didx, sg1)

        def zfill(i, _):
            for j in range(4):
                gb1[i, pl.ds(j * 16, 16)] = zero16
            return 0
        lax.fori_loop(0, CH, zfill, 0)

        def stage(i, _):
            rows = pl.ds(s * ZROWS + i * CH, CH)
            pltpu.sync_copy(gb1, acc_sh.at[rows])
            if layer1:
                pltpu.sync_copy(aux_hbm.at[0, rows], dbuf)
                pltpu.sync_copy(aux_hbm.at[1, rows], dbuf2)

                def newt(r2, _):
                    for u in range(4):
                        r = 4 * r2 + u
                        d = dbuf[r, :] + dbuf2[r, :] + 1.0
                        iv = plsc.bitcast(d, jnp.int32)
                        iv = 0x5F3759DF - lax.shift_right_logical(iv, 1)
                        y = plsc.bitcast(iv, jnp.float32)
                        for _ in range(3):
                            y = y * (1.5 - 0.5 * d * y * y)
                        dbuf[r, :] = y
                    return 0
                lax.fori_loop(0, CH // 4, newt, 0)
            else:
                pltpu.sync_copy(aux_hbm.at[rows], dbuf)
            pltpu.sync_copy(xp_hbm.at[rows], gb0)

            def scale(r2, _):
                for u in range(2):
                    r = 2 * r2 + u
                    dv = dbuf[r, :]
                    for j in range(4):
                        sl = pl.ds(j * 16, 16)
                        gb0[r, sl] = gb0[r, sl] * dv
                return 0
            lax.fori_loop(0, CH // 2, scale, 0)
            pltpu.sync_copy(gb0, xs_sh.at[rows])
            if layer1:
                @pl.when(c == 0)
                def _():
                    pltpu.sync_copy(dbuf, dinv_out.at[rows])
            return 0
        lax.fori_loop(0, ZROWS // CH, stage, 0)

        pltpu.make_async_copy(
            src_hbm.at[pl.ds(cb, NCHUNK)], sidx, sg0).wait()
        pltpu.make_async_copy(
            dst_hbm.at[pl.ds(cb, NCHUNK)], didx, sg1).wait()
        plsc.subcore_barrier()

        pltpu.async_copy(xs_sh.at[sidx.at[0]], gb0, sg0)
        pltpu.async_copy(xs_sh.at[sidx.at[1]], gb1, sg1)

        NFULL = (NCHUNK // 3) * 3

        def body_loop(g, _):
            k0 = 3 * g
            for b in range(3):
                k = k0 + b
                pltpu.make_async_copy(
                    xs_sh.at[sidx.at[k]], gbs[b], sgs[b]).wait()

                @pl.when(k + 2 < NCHUNK)
                def _():
                    pltpu.async_copy(xs_sh.at[sidx.at[k + 2]],
                                     gbs[(b + 2) % 3], sgs[(b + 2) % 3])

                pltpu.sync_copy(gbs[b], acc_sh.at[didx.at[k]], add=True)
            return 0
        lax.fori_loop(0, NFULL // 3, body_loop, 0)

        for k in range(NFULL, NCHUNK):
            b = k % 3
            pltpu.make_async_copy(
                xs_sh.at[sidx.at[k]], gbs[b], sgs[b]).wait()
            pltpu.sync_copy(gbs[b], acc_sh.at[didx.at[k]], add=True)

        plsc.subcore_barrier()

        def wb(i, _):
            rows = pl.ds(s * ZROWS + i * CH, CH)
            pltpu.sync_copy(acc_sh.at[rows], gb0)
            pltpu.sync_copy(gb0, out_hbm.at[c, rows])
            return 0
        lax.fori_loop(0, ZROWS // CH, wb, 0)
    return body


_agg1_body = _make_agg_body(True)
_agg2_body = _make_agg_body(False)

_AGG_SCRATCH = [
    pltpu.VMEM((NCHUNK, CH), jnp.int32),
    pltpu.VMEM((NCHUNK, CH), jnp.int32),
    pltpu.VMEM((CH, H), jnp.float32),
    pltpu.VMEM((CH, H), jnp.float32),
    pltpu.VMEM((CH, H), jnp.float32),
    pltpu.VMEM((CH, 16), jnp.float32),
    pltpu.VMEM((CH, 16), jnp.float32),
    pltpu.VMEM_SHARED((NPAD, H), jnp.float32),
    pltpu.VMEM_SHARED((NPAD, H), jnp.float32),
    pltpu.SemaphoreType.DMA,
    pltpu.SemaphoreType.DMA,
    pltpu.SemaphoreType.DMA,
]


def _sc_agg1(srcp, dstp, xp, degp):
    k = pl.kernel(
        _agg1_body,
        out_type=(jax.ShapeDtypeStruct((NC, NPAD, H), jnp.float32),
                  jax.ShapeDtypeStruct((NPAD, 16), jnp.float32)),
        mesh=_mesh,
        compiler_params=_sc_params,
        scratch_types=_AGG_SCRATCH,
    )
    return k(srcp, dstp, xp, degp)


def _sc_agg2(srcp, dstp, xp, dinv16):
    k = pl.kernel(
        _agg2_body,
        out_type=jax.ShapeDtypeStruct((NC, NPAD, H), jnp.float32),
        mesh=_mesh,
        compiler_params=_sc_params,
        scratch_types=_AGG_SCRATCH,
    )
    return k(srcp, dstp, xp, dinv16)


_BLK = 512
_NBLK = NPAD // _BLK


def _tc0_body(x_ref, w1_ref, xp_ref):
    xp_ref[...] = jnp.dot(x_ref[...], w1_ref[...],
                          preferred_element_type=jnp.float32)


def _tc0(xpad, W1):
    return pl.pallas_call(
        _tc0_body,
        grid=(_NBLK,),
        in_specs=[
            pl.BlockSpec((_BLK, D_IN), lambda i: (i, 0)),
            pl.BlockSpec((D_IN, H), lambda i: (0, 0)),
        ],
        out_specs=pl.BlockSpec((_BLK, H), lambda i: (i, 0)),
        out_shape=jax.ShapeDtypeStruct((NPAD, H), jnp.float32),
    )(xpad, W1)


def _tc2_body(acc_ref, xp_ref, dinv_ref, b1_ref, w2_ref, xp2_ref):
    dinv = dinv_ref[:, 0:1]
    agg = acc_ref[0] + acc_ref[1]
    h1 = jnp.maximum(dinv * agg + (dinv * dinv) * xp_ref[...] + b1_ref[...],
                     0.0)
    xp2_ref[...] = jnp.dot(h1, w2_ref[...], preferred_element_type=jnp.float32)


def _tc2(acc1, xp1, dinv16, b1, W2):
    return pl.pallas_call(
        _tc2_body,
        grid=(_NBLK,),
        in_specs=[
            pl.BlockSpec((NC, _BLK, H), lambda i: (0, i, 0)),
            pl.BlockSpec((_BLK, H), lambda i: (i, 0)),
            pl.BlockSpec((_BLK, 16), lambda i: (i, 0)),
            pl.BlockSpec((1, H), lambda i: (0, 0)),
            pl.BlockSpec((H, H), lambda i: (0, 0)),
        ],
        out_specs=pl.BlockSpec((_BLK, H), lambda i: (i, 0)),
        out_shape=jax.ShapeDtypeStruct((NPAD, H), jnp.float32),
    )(acc1, xp1, dinv16, b1, W2)


def _tc3_body(acc_ref, xp_ref, dinv_ref, b2_ref, batch_ref,
              l1w_ref, l1b_ref, l2w_ref, l2b_ref, out_ref):
    dinv = dinv_ref[:, 0:1]
    agg = acc_ref[0] + acc_ref[1]
    h2 = jnp.maximum(dinv * agg + (dinv * dinv) * xp_ref[...] + b2_ref[...],
                     0.0)
    seg = lax.broadcasted_iota(jnp.int32, (1, G), 1)
    p = (batch_ref[...] == seg).astype(jnp.float32)
    pooled_sum = lax.dot_general(
        p, h2, (((0,), (0,)), ((), ())), preferred_element_type=jnp.float32)
    counts = jnp.sum(p, axis=0)
    pooled = pooled_sum / jnp.maximum(counts, 1.0)[:, None]
    t = jnp.maximum(
        jnp.dot(pooled, l1w_ref[...], preferred_element_type=jnp.float32)
        + l1b_ref[...], 0.0)
    logits = (jnp.dot(t, l2w_ref[...], preferred_element_type=jnp.float32)
              + l2b_ref[...])
    m = jnp.max(logits, axis=1, keepdims=True)
    lse = jnp.log(jnp.sum(jnp.exp(logits - m), axis=1, keepdims=True))
    out_ref[...] = logits - m - lse


def _tc3(acc2, xp2, dinv16, b2, batchp, L1w, L1b, L2w, L2b):
    return pl.pallas_call(
        _tc3_body,
        out_shape=jax.ShapeDtypeStruct((G, C), jnp.float32),
    )(acc2, xp2, dinv16, b2, batchp, L1w, L1b, L2w, L2b)


def kernel(x, edge_index, batch, W1, b1, W2, b2, L1w, L1b, L2w, L2b):
    pad_e = EPAD - E
    srcp = jnp.concatenate(
        [edge_index[0], jnp.full((pad_e,), N, jnp.int32)]
    ).reshape(TOTCH, CH)
    dstp = jnp.concatenate(
        [edge_index[1], jnp.full((pad_e,), N, jnp.int32)]
    ).reshape(TOTCH, CH)
    xpad = jnp.pad(x, ((0, NPAD - N), (0, 0)))
    batchp = jnp.concatenate(
        [batch, jnp.full((NPAD - N,), G, jnp.int32)]).reshape(NPAD, 1)

    degp = _sc_deg(dstp)
    xp1 = _tc0(xpad, W1)
    acc1, dinv16 = _sc_agg1(srcp, dstp, xp1, degp)
    xp2 = _tc2(acc1, xp1, dinv16, b1.reshape(1, H), W2)
    acc2 = _sc_agg2(srcp, dstp, xp2, dinv16)
    return _tc3(acc2, xp2, dinv16, b2.reshape(1, H), batchp,
                L1w, L1b.reshape(1, 32), L2w, L2b.reshape(1, C))
